# baseline (speedup 1.0000x reference)
import jax
import jax.numpy as jnp
from jax import lax
from jax.experimental import pallas as pl
from jax.experimental.pallas import tpu as pltpu
from jax.experimental.pallas import tpu_sc as plsc

_NC = 2
_NS = 16
_NW = _NC * _NS
_B = 128
_P = 32768
_K = _P // 4
_RPW = _B // _NW
_CHUNKS = _P // 16


def _last_true(bools):
    pc = plsc.all_reduce_population_count(bools)
    if pc.ndim:
        pc = pc[0]
    return pc - jnp.int32(1)


def _pick_bucket(hist, lanes, need):
    zeros = jnp.zeros((16,), jnp.int32)
    gtot = zeros
    for g in range(16):
        gtot = jnp.where(lanes == g, jnp.sum(hist[pl.ds(g * 16, 16)]), gtot)
    sincl_g = lax.rev(plsc.cumsum(lax.rev(gtot, (0,))), (0,))
    grp = _last_true(sincl_g >= need)
    tot_grp = jnp.sum(jnp.where(lanes == grp, gtot, 0))
    s_grp = jnp.sum(jnp.where(lanes == grp, sincl_g, 0))
    above_grp = s_grp - tot_grp

    h = hist[pl.ds(grp * 16, 16)]
    s_in = lax.rev(plsc.cumsum(lax.rev(h, (0,))), (0,)) + above_grp
    b15 = _last_true(s_in >= need)
    cnt_b = jnp.sum(jnp.where(lanes == b15, h, 0))
    s_b = jnp.sum(jnp.where(lanes == b15, s_in, 0))
    return grp * 16 + b15, s_b - cnt_b, cnt_b


def _sc_body(loss_hbm, out_hbm, rowbuf0, rowbuf1, hist, fsum, outv, sem0, sem1):
    wid = lax.axis_index("s") * _NC + lax.axis_index("c")
    lanes = lax.iota(jnp.int32, 16)
    ones = jnp.ones((16,), jnp.int32)
    zeros = jnp.zeros((16,), jnp.int32)
    fzeros = jnp.zeros((16,), jnp.float32)

    bufs = [rowbuf0, rowbuf1]
    sems = [sem0, sem1]
    base = wid * _RPW
    copies = [pltpu.async_copy(loss_hbm.at[base], rowbuf0, sem0), None]

    sums_vec = fzeros
    for j in range(_RPW):
        rowbuf = bufs[j % 2]
        if j + 1 < _RPW:
            copies[(j + 1) % 2] = pltpu.async_copy(
                loss_hbm.at[base + j + 1], bufs[(j + 1) % 2], sems[(j + 1) % 2]
            )
        copies[j % 2].wait()

        need = jnp.int32(_K)
        prefix = jnp.uint32(0)
        for lvl in range(3):
            shift = jnp.uint32(24 - 8 * lvl)

            for c in range(17):
                hist[pl.ds(c * 16, 16)] = zeros

            @plsc.parallel_loop(0, _CHUNKS, unroll=16)
            def hist_fn(c, lvl=lvl, prefix=prefix, shift=shift, rowbuf=rowbuf):
                key = lax.bitcast_convert_type(rowbuf[pl.ds(c * 16, 16)], jnp.uint32)
                byte = ((key >> shift) & jnp.uint32(0xFF)).astype(jnp.int32)
                if lvl == 0:
                    plsc.addupdate_scatter(hist, [byte], ones)
                else:
                    m = (key >> (shift + jnp.uint32(8))) == prefix
                    plsc.addupdate_scatter(hist, [byte], ones, mask=m)

            bkt, above, _ = _pick_bucket(hist, lanes, need)
            need = need - above
            prefix = (prefix << jnp.uint32(8)) | bkt.astype(jnp.uint32)

        for c in range(17):
            hist[pl.ds(c * 16, 16)] = zeros
            fsum[pl.ds(c * 16, 16)] = fzeros

        def lvl3_fn(c, accv, prefix=prefix, rowbuf=rowbuf):
            v = rowbuf[pl.ds(c * 16, 16)]
            key = lax.bitcast_convert_type(v, jnp.uint32)
            hi24 = key >> jnp.uint32(8)
            m = hi24 == prefix
            byte = (key & jnp.uint32(0xFF)).astype(jnp.int32)
            plsc.addupdate_scatter(hist, [byte], ones, mask=m)
            plsc.addupdate_scatter(fsum, [byte], v, mask=m)
            return accv + jnp.where(hi24 > prefix, v, jnp.float32(0.0))

        accv = plsc.parallel_loop(0, _CHUNKS, unroll=16, carry=fzeros)(lvl3_fn)
        gt_sum = jnp.sum(accv)

        bkt, above, _ = _pick_bucket(hist, lanes, need)
        need = need - above
        tkey = (prefix << jnp.uint32(8)) | bkt.astype(jnp.uint32)

        grp = bkt // 16
        b15 = bkt % 16
        fg = fzeros
        for g in range(16):
            fg = jnp.where(lanes == g, jnp.sum(fsum[pl.ds(g * 16, 16)]), fg)
        fincl_g = lax.rev(plsc.cumsum(lax.rev(fg, (0,))), (0,))
        f_grp = jnp.sum(jnp.where(lanes == grp, fg, 0.0))
        fs_grp = jnp.sum(jnp.where(lanes == grp, fincl_g, 0.0))
        fh = fsum[pl.ds(grp * 16, 16)]
        fs_in = lax.rev(plsc.cumsum(lax.rev(fh, (0,))), (0,)) + (fs_grp - f_grp)
        fs_b = jnp.sum(jnp.where(lanes == b15, fs_in, 0.0))
        f_b = jnp.sum(jnp.where(lanes == b15, fh, 0.0))
        fsum_suffix = fs_b - f_b

        tval = lax.bitcast_convert_type(jnp.full((16,), tkey, jnp.uint32), jnp.float32)[0]
        rowsum = gt_sum + fsum_suffix + need.astype(jnp.float32) * tval
        sums_vec = jnp.where(lanes == j, rowsum, sums_vec)

    outv[...] = sums_vec
    pltpu.sync_copy(outv, out_hbm.at[wid])


@jax.jit
def kernel(loss, dummy):
    b = loss.shape[0]
    loss = loss.reshape(b, -1)
    mesh = plsc.VectorSubcoreMesh(core_axis_name="c", subcore_axis_name="s")
    sums = pl.kernel(
        _sc_body,
        mesh=mesh,
        out_type=jax.ShapeDtypeStruct((_NW, 16), jnp.float32),
        compiler_params=pltpu.CompilerParams(needs_layout_passes=False),
        scratch_types=[
            pltpu.VMEM((_P,), jnp.float32),
            pltpu.VMEM((_P,), jnp.float32),
            pltpu.VMEM((272,), jnp.int32),
            pltpu.VMEM((272,), jnp.float32),
            pltpu.VMEM((16,), jnp.float32),
            pltpu.SemaphoreType.DMA,
            pltpu.SemaphoreType.DMA,
        ],
    )(loss)
    return jnp.sum(sums) / (_B * _K)


# speedup vs baseline: 1.3077x; 1.3077x over previous
import jax
import jax.numpy as jnp
from jax import lax
from jax.experimental import pallas as pl
from jax.experimental.pallas import tpu as pltpu
from jax.experimental.pallas import tpu_sc as plsc

_NC = 2
_NS = 16
_NW = _NC * _NS
_B = 128
_P = 32768
_K = _P // 4
_RPW = _B // _NW
_CHUNKS = _P // 16


def _last_true(bools):
    pc = plsc.all_reduce_population_count(bools)
    if pc.ndim:
        pc = pc[0]
    return pc - jnp.int32(1)


def _pick_bucket(hist, lanes, need):
    zeros = jnp.zeros((16,), jnp.int32)
    gtot = zeros
    for g in range(16):
        gtot = jnp.where(lanes == g, jnp.sum(hist[pl.ds(g * 16, 16)]), gtot)
    sincl_g = lax.rev(plsc.cumsum(lax.rev(gtot, (0,))), (0,))
    grp = _last_true(sincl_g >= need)
    tot_grp = jnp.sum(jnp.where(lanes == grp, gtot, 0))
    s_grp = jnp.sum(jnp.where(lanes == grp, sincl_g, 0))
    above_grp = s_grp - tot_grp

    h = hist[pl.ds(grp * 16, 16)]
    s_in = lax.rev(plsc.cumsum(lax.rev(h, (0,))), (0,)) + above_grp
    b15 = _last_true(s_in >= need)
    cnt_b = jnp.sum(jnp.where(lanes == b15, h, 0))
    s_b = jnp.sum(jnp.where(lanes == b15, s_in, 0))
    return grp * 16 + b15, s_b - cnt_b, cnt_b


def _sc_body(loss_hbm, out_hbm, rowbuf0, rowbuf1, hist, hist4, fsum, outv, sem0, sem1):
    wid = lax.axis_index("s") * _NC + lax.axis_index("c")
    lanes = lax.iota(jnp.int32, 16)
    ones = jnp.ones((16,), jnp.int32)
    zeros = jnp.zeros((16,), jnp.int32)
    fzeros = jnp.zeros((16,), jnp.float32)

    bufs = [rowbuf0, rowbuf1]
    sems = [sem0, sem1]
    base = wid * _RPW
    copies = [pltpu.async_copy(loss_hbm.at[base], rowbuf0, sem0), None]

    sums_vec = fzeros
    for j in range(_RPW):
        rowbuf = bufs[j % 2]
        if j + 1 < _RPW:
            copies[(j + 1) % 2] = pltpu.async_copy(
                loss_hbm.at[base + j + 1], bufs[(j + 1) % 2], sems[(j + 1) % 2]
            )
        copies[j % 2].wait()

        need = jnp.int32(_K)
        prefix = jnp.uint32(0)
        for lvl in range(3):
            shift = jnp.uint32(24 - 8 * lvl)

            if lvl == 0:
                for c in range(65):
                    hist4[pl.ds(c * 16, 16)] = zeros

                @plsc.parallel_loop(0, _CHUNKS, unroll=8)
                def hist0_fn(c, rowbuf=rowbuf):
                    key = lax.bitcast_convert_type(rowbuf[pl.ds(c * 16, 16)], jnp.uint32)
                    byte = (key >> jnp.uint32(24)).astype(jnp.int32)
                    plsc.addupdate_scatter(hist4, [byte * 4 + (lanes & 3)], ones)

                for g in range(16):
                    acc = zeros
                    for s_ in range(4):
                        acc = acc + plsc.load_gather(hist4, [(g * 16 + lanes) * 4 + s_])
                    hist[pl.ds(g * 16, 16)] = acc
            else:
                for c in range(17):
                    hist[pl.ds(c * 16, 16)] = zeros

                @plsc.parallel_loop(0, _CHUNKS, unroll=8)
                def hist_fn(c, prefix=prefix, shift=shift, rowbuf=rowbuf):
                    key = lax.bitcast_convert_type(rowbuf[pl.ds(c * 16, 16)], jnp.uint32)
                    byte = ((key >> shift) & jnp.uint32(0xFF)).astype(jnp.int32)
                    m = (key >> (shift + jnp.uint32(8))) == prefix
                    plsc.addupdate_scatter(hist, [byte], ones, mask=m)

            bkt, above, _ = _pick_bucket(hist, lanes, need)
            need = need - above
            prefix = (prefix << jnp.uint32(8)) | bkt.astype(jnp.uint32)

        for c in range(17):
            hist[pl.ds(c * 16, 16)] = zeros
            fsum[pl.ds(c * 16, 16)] = fzeros

        def lvl3_fn(c, accv, prefix=prefix, rowbuf=rowbuf):
            v = rowbuf[pl.ds(c * 16, 16)]
            key = lax.bitcast_convert_type(v, jnp.uint32)
            hi24 = key >> jnp.uint32(8)
            m = hi24 == prefix
            byte = (key & jnp.uint32(0xFF)).astype(jnp.int32)
            plsc.addupdate_scatter(hist, [byte], ones, mask=m)
            plsc.addupdate_scatter(fsum, [byte], v, mask=m)
            return accv + jnp.where(hi24 > prefix, v, jnp.float32(0.0))

        accv = plsc.parallel_loop(0, _CHUNKS, unroll=8, carry=fzeros)(lvl3_fn)
        gt_sum = jnp.sum(accv)

        bkt, above, _ = _pick_bucket(hist, lanes, need)
        need = need - above
        tkey = (prefix << jnp.uint32(8)) | bkt.astype(jnp.uint32)

        grp = bkt // 16
        b15 = bkt % 16
        fg = fzeros
        for g in range(16):
            fg = jnp.where(lanes == g, jnp.sum(fsum[pl.ds(g * 16, 16)]), fg)
        fincl_g = lax.rev(plsc.cumsum(lax.rev(fg, (0,))), (0,))
        f_grp = jnp.sum(jnp.where(lanes == grp, fg, 0.0))
        fs_grp = jnp.sum(jnp.where(lanes == grp, fincl_g, 0.0))
        fh = fsum[pl.ds(grp * 16, 16)]
        fs_in = lax.rev(plsc.cumsum(lax.rev(fh, (0,))), (0,)) + (fs_grp - f_grp)
        fs_b = jnp.sum(jnp.where(lanes == b15, fs_in, 0.0))
        f_b = jnp.sum(jnp.where(lanes == b15, fh, 0.0))
        fsum_suffix = fs_b - f_b

        tval = lax.bitcast_convert_type(jnp.full((16,), tkey, jnp.uint32), jnp.float32)[0]
        rowsum = gt_sum + fsum_suffix + need.astype(jnp.float32) * tval
        sums_vec = jnp.where(lanes == j, rowsum, sums_vec)

    outv[...] = sums_vec
    pltpu.sync_copy(outv, out_hbm.at[wid])


@jax.jit
def kernel(loss, dummy):
    b = loss.shape[0]
    loss = loss.reshape(b, -1)
    mesh = plsc.VectorSubcoreMesh(core_axis_name="c", subcore_axis_name="s")
    sums = pl.kernel(
        _sc_body,
        mesh=mesh,
        out_type=jax.ShapeDtypeStruct((_NW, 16), jnp.float32),
        compiler_params=pltpu.CompilerParams(needs_layout_passes=False),
        scratch_types=[
            pltpu.VMEM((_P,), jnp.float32),
            pltpu.VMEM((_P,), jnp.float32),
            pltpu.VMEM((272,), jnp.int32),
            pltpu.VMEM((1056,), jnp.int32),
            pltpu.VMEM((272,), jnp.float32),
            pltpu.VMEM((16,), jnp.float32),
            pltpu.SemaphoreType.DMA,
            pltpu.SemaphoreType.DMA,
        ],
    )(loss)
    return jnp.sum(sums) / (_B * _K)
